# baseline (speedup 1.0000x reference)
import functools

import jax
import jax.numpy as jnp
from jax import lax
from jax.experimental import pallas as pl
from jax.experimental.pallas import tpu as pltpu
from jax.experimental.pallas import tpu_sc as plsc

S = 256
F = 2048
NEAR = 0.1
FAR = 100.0
EPS = 1e-8
NCOEF = 48
BAND = 32
S2 = 272
BIGF = 1e30
CLIP = 1e18


def _coef_body(v_ref, c_ref):
    X0 = v_ref[0]; X1 = v_ref[1]; X2 = v_ref[2]
    Y0 = v_ref[3]; Y1 = v_ref[4]; Y2 = v_ref[5]
    Z0 = v_ref[6]; Z1 = v_ref[7]; Z2 = v_ref[8]
    a0 = Y1 - Y2; b0 = X2 - X1; c0 = X1 * Y2 - X2 * Y1
    a1 = Y2 - Y0; b1 = X0 - X2; c1 = X2 * Y0 - X0 * Y2
    a2 = Y0 - Y1; b2 = X1 - X0; c2 = X0 * Y1 - X1 * Y0
    det = c0 + c1 + c2
    sgn = jnp.where(det >= 0.0, 1.0, -1.0)
    adet = jnp.abs(det)
    valid = adet > EPS
    iZ0 = 1.0 / jnp.where(jnp.abs(Z0) > EPS, Z0, 1.0)
    iZ1 = 1.0 / jnp.where(jnp.abs(Z1) > EPS, Z1, 1.0)
    iZ2 = 1.0 / jnp.where(jnp.abs(Z2) > EPS, Z2, 1.0)
    ga = (a0 * iZ0 + a1 * iZ1 + a2 * iZ2) * sgn
    gb = (b0 * iZ0 + b1 * iZ1 + b2 * iZ2) * sgn
    gc = (c0 * iZ0 + c1 * iZ1 + c2 * iZ2) * sgn
    a0 = a0 * sgn; b0 = b0 * sgn; c0 = c0 * sgn
    a1 = a1 * sgn; b1 = b1 * sgn; c1 = c1 * sgn
    a2 = a2 * sgn; b2 = b2 * sgn; c2 = c2 * sgn
    glo = jnp.where(valid, adet * (1.0 / FAR), BIGF)
    ghi = jnp.where(valid, adet * (1.0 / NEAR), -BIGF)
    c_ref[0] = a0
    c_ref[1] = b0
    c_ref[2] = c0
    c_ref[3] = a1
    c_ref[4] = b1
    c_ref[5] = c1
    c_ref[6] = a2
    c_ref[7] = b2
    c_ref[8] = c2
    c_ref[9] = ga
    c_ref[10] = gb
    c_ref[11] = gc
    c_ref[12] = jnp.where(valid, 1.0 / adet, 0.0)
    c_ref[13] = glo
    c_ref[14] = ghi
    c_ref[15] = jnp.zeros_like(det)
    cons = (
        (a0, -b0 / a0, -c0 / a0),
        (a1, -b1 / a1, -c1 / a1),
        (a2, -b2 / a2, -c2 / a2),
        (ga, -gb / ga, (glo - gc) / ga),
        (-ga, -gb / ga, (ghi - gc) / ga),
    )
    z = jnp.zeros_like(det)
    for k, (A, p, q) in enumerate(cons):
        pc = jnp.clip(p, -CLIP, CLIP)
        qc = jnp.clip(q, -CLIP, CLIP)
        pc = jnp.where(jnp.isnan(pc), 0.0, pc)
        qc = jnp.where(jnp.isnan(qc), 0.0, qc)
        pos = A > 0.0
        neg = A < 0.0
        c_ref[16 + k] = jnp.where(pos, pc, z)
        c_ref[21 + k] = jnp.where(pos, qc, -BIGF)
        c_ref[32 + k] = jnp.where(neg, pc, z)
        c_ref[37 + k] = jnp.where(neg, qc, BIGF)
    for i in (26, 27, 28, 29, 30, 31, 42, 43, 44, 45, 46, 47):
        c_ref[i] = z


def _sc_raster_body(coef_hbm, out_hbm, cvm, den, idx):
    cid = lax.axis_index("c")
    sid = lax.axis_index("s")
    wid = sid * 2 + cid
    b = wid >> 3
    rbase = wid & 7

    pltpu.sync_copy(coef_hbm.at[b], cvm)

    def _init(r, _):
        for k in range(S2 // 16):
            col = k * 16
            den[r, pl.ds(col, 16)] = jnp.full((16,), 1.0 / FAR, jnp.float32)
            idx[r, pl.ds(col, 16)] = jnp.full((16,), -1, jnp.int32)
        return 0
    lax.fori_loop(0, BAND, _init, 0)

    lane = lax.iota(jnp.int32, 16)
    lane_f = lane.astype(jnp.float32)
    rbase_f = rbase.astype(jnp.float32)
    inv_s = jnp.float32(1.0 / S)

    def face_body(f, _):
        v1 = cvm[pl.ds(pl.multiple_of(f * NCOEF, 16), 16)]
        v2 = cvm[pl.ds(pl.multiple_of(f * NCOEF + 16, 16), 16)]
        v3 = cvm[pl.ds(pl.multiple_of(f * NCOEF + 32, 16), 16)]
        a0 = v1[0]; b0 = v1[1]; c0 = v1[2]
        a1 = v1[3]; b1 = v1[4]; c1 = v1[5]
        a2 = v1[6]; b2 = v1[7]; c2 = v1[8]
        ga = v1[9]; gb = v1[10]; gc = v1[11]
        radet = v1[12]; glo = v1[13]; ghi = v1[14]

        for ch in range(BAND // 16):
            yv = (2.0 * (rbase_f + 8.0 * (lane_f + (ch * 16.0)))
                  + (1.0 - S)) * inv_s
            xlo = v2[5] + v2[0] * yv
            xhi = v3[5] + v3[0] * yv
            for k in range(1, 5):
                xlo = jnp.maximum(xlo, v2[5 + k] + v2[k] * yv)
                xhi = jnp.minimum(xhi, v3[5 + k] + v3[k] * yv)
            qjl = jnp.clip((S * xlo + (S - 1.0)) * 0.5, -2.0, 300.0)
            qjh = jnp.clip((S * xhi + (S - 1.0)) * 0.5, -2.0, 300.0)
            jlv = jnp.maximum(qjl.astype(jnp.int32) - 1, 0)
            jhv = jnp.minimum(qjh.astype(jnp.int32) + 1, S - 1)
            b0rv = b0 * yv + c0
            b1rv = b1 * yv + c1
            b2rv = b2 * yv + c2
            bgrv = gb * yv + gc

            for h in range(16):
                jl = jlv[h]
                jh = jhv[h]

                @pl.when(jl <= jh)
                def _do_row(ch=ch, h=h, jl=jl, jh=jh):
                    trow = ch * 16 + h
                    b0r = b0rv[h]; b1r = b1rv[h]; b2r = b2rv[h]
                    bgr = bgrv[h]
                    base = jl & (-16)
                    nch = ((jh - base) >> 5) + 1

                    @plsc.parallel_loop(0, nch)
                    def ch_body(k):
                        c32 = base + k * 32
                        for hh in range(2):
                            col = pl.multiple_of(c32 + hh * 16, 16)
                            iv = lane + col
                            xv = (2.0 * iv.astype(jnp.float32)
                                  + (1.0 - S)) * inv_s
                            w0 = a0 * xv + b0r
                            w1 = a1 * xv + b1r
                            w2 = a2 * xv + b2r
                            g = ga * xv + bgr
                            dn = g * radet
                            dold = den[trow, pl.ds(col, 16)]
                            m = ((w0 >= 0.0) & (w1 >= 0.0) & (w2 >= 0.0)
                                 & (g > glo) & (g < ghi) & (dn > dold))
                            if hh == 1:
                                m = m & (iv < S)
                            den[trow, pl.ds(col, 16)] = jnp.where(m, dn, dold)
                            iold = idx[trow, pl.ds(col, 16)]
                            idx[trow, pl.ds(col, 16)] = jnp.where(m, f, iold)

        return 0

    lax.fori_loop(0, F, face_body, 0)

    pltpu.sync_copy(idx.at[:, pl.ds(0, S)], out_hbm.at[b, rbase])


def kernel(inputs):
    B = inputs.shape[0]
    v = jnp.transpose(inputs, (3, 2, 0, 1)).reshape(9, B, F)
    coef = pl.pallas_call(
        _coef_body,
        out_shape=jax.ShapeDtypeStruct((NCOEF, B, F), jnp.float32),
    )(v)
    coef = jnp.transpose(coef, (1, 2, 0)).reshape(B, F * NCOEF)

    mesh = plsc.VectorSubcoreMesh(core_axis_name="c", subcore_axis_name="s")
    sc = functools.partial(
        pl.kernel,
        mesh=mesh,
        out_type=jax.ShapeDtypeStruct((B, 8, BAND, S), jnp.int32),
        scratch_types=[
            pltpu.VMEM((F * NCOEF,), jnp.float32),
            pltpu.VMEM((BAND, S2), jnp.float32),
            pltpu.VMEM((BAND, S2), jnp.int32),
        ],
    )(_sc_raster_body)
    out_perm = sc(coef)
    return jnp.transpose(out_perm, (0, 2, 1, 3)).reshape(B, S, S)


# speedup vs baseline: 4.1927x; 4.1927x over previous
import functools

import jax
import jax.numpy as jnp
from jax import lax
from jax.experimental import pallas as pl
from jax.experimental.pallas import tpu as pltpu
from jax.experimental.pallas import tpu_sc as plsc

S = 256
F = 2048
NEAR = 0.1
FAR = 100.0
EPS = 1e-8
NCOEF = 48
BAND = 32
S2 = 272
BIGF = 1e30
CLIP = 1e18


def _coef_body(v_ref, c_ref):
    X0 = v_ref[0]; X1 = v_ref[1]; X2 = v_ref[2]
    Y0 = v_ref[3]; Y1 = v_ref[4]; Y2 = v_ref[5]
    Z0 = v_ref[6]; Z1 = v_ref[7]; Z2 = v_ref[8]
    a0 = Y1 - Y2; b0 = X2 - X1; c0 = X1 * Y2 - X2 * Y1
    a1 = Y2 - Y0; b1 = X0 - X2; c1 = X2 * Y0 - X0 * Y2
    a2 = Y0 - Y1; b2 = X1 - X0; c2 = X0 * Y1 - X1 * Y0
    det = c0 + c1 + c2
    sgn = jnp.where(det >= 0.0, 1.0, -1.0)
    adet = jnp.abs(det)
    valid = adet > EPS
    iZ0 = 1.0 / jnp.where(jnp.abs(Z0) > EPS, Z0, 1.0)
    iZ1 = 1.0 / jnp.where(jnp.abs(Z1) > EPS, Z1, 1.0)
    iZ2 = 1.0 / jnp.where(jnp.abs(Z2) > EPS, Z2, 1.0)
    ga = (a0 * iZ0 + a1 * iZ1 + a2 * iZ2) * sgn
    gb = (b0 * iZ0 + b1 * iZ1 + b2 * iZ2) * sgn
    gc = (c0 * iZ0 + c1 * iZ1 + c2 * iZ2) * sgn
    a0 = a0 * sgn; b0 = b0 * sgn; c0 = c0 * sgn
    a1 = a1 * sgn; b1 = b1 * sgn; c1 = c1 * sgn
    a2 = a2 * sgn; b2 = b2 * sgn; c2 = c2 * sgn
    glo = jnp.where(valid, adet * (1.0 / FAR), BIGF)
    ghi = jnp.where(valid, adet * (1.0 / NEAR), -BIGF)
    c_ref[0] = a0
    c_ref[1] = b0
    c_ref[2] = c0
    c_ref[3] = a1
    c_ref[4] = b1
    c_ref[5] = c1
    c_ref[6] = a2
    c_ref[7] = b2
    c_ref[8] = c2
    c_ref[9] = ga
    c_ref[10] = gb
    c_ref[11] = gc
    c_ref[12] = jnp.where(valid, 1.0 / adet, 0.0)
    c_ref[13] = glo
    c_ref[14] = ghi
    ymin = jnp.minimum(jnp.minimum(Y0, Y1), Y2)
    ymax = jnp.maximum(jnp.maximum(Y0, Y1), Y2)
    qlo = jnp.clip((S * ymin + (S - 1.0)) * 0.5, -2.0, 300.0)
    qhi = jnp.clip((S * ymax + (S - 1.0)) * 0.5, -2.0, 300.0)
    c_ref[15] = jnp.where(valid, qlo, 300.0)
    c_ref[26] = jnp.where(valid, qhi, -2.0)
    cons = (
        (a0, -b0 / a0, -c0 / a0),
        (a1, -b1 / a1, -c1 / a1),
        (a2, -b2 / a2, -c2 / a2),
        (ga, -gb / ga, (glo - gc) / ga),
        (-ga, -gb / ga, (ghi - gc) / ga),
    )
    z = jnp.zeros_like(det)
    for k, (A, p, q) in enumerate(cons):
        pc = jnp.clip(p, -CLIP, CLIP)
        qc = jnp.clip(q, -CLIP, CLIP)
        pc = jnp.where(jnp.isnan(pc), 0.0, pc)
        qc = jnp.where(jnp.isnan(qc), 0.0, qc)
        pos = A > 0.0
        neg = A < 0.0
        c_ref[16 + k] = jnp.where(pos, pc, z)
        c_ref[21 + k] = jnp.where(pos, qc, -BIGF)
        c_ref[32 + k] = jnp.where(neg, pc, z)
        c_ref[37 + k] = jnp.where(neg, qc, BIGF)
    for i in (27, 28, 29, 30, 31, 42, 43, 44, 45, 46, 47):
        c_ref[i] = z


def _sc_raster_body(coef_hbm, out_hbm, cvm, den, idx):
    cid = lax.axis_index("c")
    sid = lax.axis_index("s")
    wid = sid * 2 + cid
    b = wid >> 3
    rbase = wid & 7

    pltpu.sync_copy(coef_hbm.at[b], cvm)

    def _init(r, _):
        for k in range(S2 // 16):
            col = k * 16
            den[r, pl.ds(col, 16)] = jnp.full((16,), 1.0 / FAR, jnp.float32)
            idx[r, pl.ds(col, 16)] = jnp.full((16,), -1, jnp.int32)
        return 0
    lax.fori_loop(0, BAND, _init, 0)

    lane = lax.iota(jnp.int32, 16)
    lane_f = lane.astype(jnp.float32)
    rbase_f = rbase.astype(jnp.float32)
    inv_s = jnp.float32(1.0 / S)

    def face_body(f, _):
        v1 = cvm[pl.ds(pl.multiple_of(f * NCOEF, 16), 16)]
        v2 = cvm[pl.ds(pl.multiple_of(f * NCOEF + 16, 16), 16)]
        v3 = cvm[pl.ds(pl.multiple_of(f * NCOEF + 32, 16), 16)]
        a0 = v1[0]; b0 = v1[1]; c0 = v1[2]
        a1 = v1[3]; b1 = v1[4]; c1 = v1[5]
        a2 = v1[6]; b2 = v1[7]; c2 = v1[8]
        ga = v1[9]; gb = v1[10]; gc = v1[11]
        radet = v1[12]; glo = v1[13]; ghi = v1[14]
        qlo = v1[15]; qhi = v2[10]

        rlo_g = jnp.maximum(qlo.astype(jnp.int32) - 1, 0)
        rhi_g = jnp.minimum(qhi.astype(jnp.int32) + 1, S - 1)
        tlo = jnp.maximum((rlo_g - rbase + 7) >> 3, 0)
        thi = jnp.minimum((rhi_g - rbase) >> 3, BAND - 1)

        @pl.when(tlo <= thi)
        def _do_face():
            @plsc.parallel_loop(tlo, thi + 1)
            def row_body(r):
                yr = (2.0 * (rbase_f + 8.0 * r.astype(jnp.float32))
                      + (1.0 - S)) * inv_s
                xlo = v2[5] + v2[0] * yr
                xhi = v3[5] + v3[0] * yr
                xlo = jnp.maximum(xlo, v2[6] + v2[1] * yr)
                xhi = jnp.minimum(xhi, v3[6] + v3[1] * yr)
                xlo = jnp.maximum(xlo, v2[7] + v2[2] * yr)
                xhi = jnp.minimum(xhi, v3[7] + v3[2] * yr)
                xlo = jnp.maximum(xlo, v2[8] + v2[3] * yr)
                xhi = jnp.minimum(xhi, v3[8] + v3[3] * yr)
                xlo = jnp.maximum(xlo, v2[9] + v2[4] * yr)
                xhi = jnp.minimum(xhi, v3[9] + v3[4] * yr)
                qjl = jnp.clip((S * xlo + (S - 1.0)) * 0.5, -2.0, 300.0)
                qjh = jnp.clip((S * xhi + (S - 1.0)) * 0.5, -2.0, 300.0)
                jl = jnp.maximum(qjl.astype(jnp.int32) - 1, 0)
                jh = jnp.minimum(qjh.astype(jnp.int32) + 1, S - 1)

                @pl.when(jl <= jh)
                def _do_row():
                    b0r = b0 * yr + c0
                    b1r = b1 * yr + c1
                    b2r = b2 * yr + c2
                    bgr = gb * yr + gc
                    base = jl & (-16)
                    nch = ((jh - base) >> 5) + 1

                    @plsc.parallel_loop(0, nch)
                    def ch_body(k):
                        c32 = base + k * 32
                        for h in range(2):
                            col = pl.multiple_of(c32 + h * 16, 16)
                            iv = lane + col
                            xv = (2.0 * iv.astype(jnp.float32)
                                  + (1.0 - S)) * inv_s
                            w0 = a0 * xv + b0r
                            w1 = a1 * xv + b1r
                            w2 = a2 * xv + b2r
                            g = ga * xv + bgr
                            dn = g * radet
                            dold = den[r, pl.ds(col, 16)]
                            m = ((w0 >= 0.0) & (w1 >= 0.0) & (w2 >= 0.0)
                                 & (g > glo) & (g < ghi) & (dn > dold))
                            if h == 1:
                                m = m & (iv < S)
                            den[r, pl.ds(col, 16)] = jnp.where(m, dn, dold)
                            iold = idx[r, pl.ds(col, 16)]
                            idx[r, pl.ds(col, 16)] = jnp.where(m, f, iold)

        return 0

    lax.fori_loop(0, F, face_body, 0)

    pltpu.sync_copy(idx.at[:, pl.ds(0, S)], out_hbm.at[b, rbase])


def kernel(inputs):
    B = inputs.shape[0]
    v = jnp.transpose(inputs, (3, 2, 0, 1)).reshape(9, B, F)
    coef = pl.pallas_call(
        _coef_body,
        out_shape=jax.ShapeDtypeStruct((NCOEF, B, F), jnp.float32),
    )(v)
    coef = jnp.transpose(coef, (1, 2, 0)).reshape(B, F * NCOEF)

    mesh = plsc.VectorSubcoreMesh(core_axis_name="c", subcore_axis_name="s")
    sc = functools.partial(
        pl.kernel,
        mesh=mesh,
        out_type=jax.ShapeDtypeStruct((B, 8, BAND, S), jnp.int32),
        scratch_types=[
            pltpu.VMEM((F * NCOEF,), jnp.float32),
            pltpu.VMEM((BAND, S2), jnp.float32),
            pltpu.VMEM((BAND, S2), jnp.int32),
        ],
    )(_sc_raster_body)
    out_perm = sc(coef)
    return jnp.transpose(out_perm, (0, 2, 1, 3)).reshape(B, S, S)


# speedup vs baseline: 5.3716x; 1.2812x over previous
import functools

import jax
import jax.numpy as jnp
from jax import lax
from jax.experimental import pallas as pl
from jax.experimental.pallas import tpu as pltpu
from jax.experimental.pallas import tpu_sc as plsc

S = 256
F = 2048
NEAR = 0.1
FAR = 100.0
EPS = 1e-8
NCOEF = 48
BAND = 32
S2 = 272
BIGF = 1e30
CLIP = 1e18


def _coef_body(v_ref, c_ref):
    X0 = v_ref[0]; X1 = v_ref[1]; X2 = v_ref[2]
    Y0 = v_ref[3]; Y1 = v_ref[4]; Y2 = v_ref[5]
    Z0 = v_ref[6]; Z1 = v_ref[7]; Z2 = v_ref[8]
    a0 = Y1 - Y2; b0 = X2 - X1; c0 = X1 * Y2 - X2 * Y1
    a1 = Y2 - Y0; b1 = X0 - X2; c1 = X2 * Y0 - X0 * Y2
    a2 = Y0 - Y1; b2 = X1 - X0; c2 = X0 * Y1 - X1 * Y0
    det = c0 + c1 + c2
    sgn = jnp.where(det >= 0.0, 1.0, -1.0)
    adet = jnp.abs(det)
    valid = adet > EPS
    iZ0 = 1.0 / jnp.where(jnp.abs(Z0) > EPS, Z0, 1.0)
    iZ1 = 1.0 / jnp.where(jnp.abs(Z1) > EPS, Z1, 1.0)
    iZ2 = 1.0 / jnp.where(jnp.abs(Z2) > EPS, Z2, 1.0)
    ga = (a0 * iZ0 + a1 * iZ1 + a2 * iZ2) * sgn
    gb = (b0 * iZ0 + b1 * iZ1 + b2 * iZ2) * sgn
    gc = (c0 * iZ0 + c1 * iZ1 + c2 * iZ2) * sgn
    a0 = a0 * sgn; b0 = b0 * sgn; c0 = c0 * sgn
    a1 = a1 * sgn; b1 = b1 * sgn; c1 = c1 * sgn
    a2 = a2 * sgn; b2 = b2 * sgn; c2 = c2 * sgn
    glo = jnp.where(valid, adet * (1.0 / FAR), BIGF)
    ghi = jnp.where(valid, adet * (1.0 / NEAR), -BIGF)
    c_ref[0] = a0
    c_ref[1] = b0
    c_ref[2] = c0
    c_ref[3] = a1
    c_ref[4] = b1
    c_ref[5] = c1
    c_ref[6] = a2
    c_ref[7] = b2
    c_ref[8] = c2
    c_ref[9] = ga
    c_ref[10] = gb
    c_ref[11] = gc
    c_ref[12] = jnp.where(valid, 1.0 / adet, 0.0)
    c_ref[13] = glo
    c_ref[14] = ghi
    cons = (
        (a0, -b0 / a0, -c0 / a0),
        (a1, -b1 / a1, -c1 / a1),
        (a2, -b2 / a2, -c2 / a2),
        (ga, -gb / ga, (glo - gc) / ga),
        (-ga, -gb / ga, (ghi - gc) / ga),
    )
    z = jnp.zeros_like(det)
    for k, (A, p, q) in enumerate(cons):
        pc = jnp.clip(p, -CLIP, CLIP)
        qc = jnp.clip(q, -CLIP, CLIP)
        pc = jnp.where(jnp.isnan(pc), 0.0, pc)
        qc = jnp.where(jnp.isnan(qc), 0.0, qc)
        pos = A > 0.0
        neg = A < 0.0
        c_ref[16 + k] = jnp.where(pos, pc, z)
        c_ref[21 + k] = jnp.where(pos, qc, -BIGF)
        c_ref[32 + k] = jnp.where(neg, pc, z)
        c_ref[37 + k] = jnp.where(neg, qc, BIGF)
    pL = []; qL = []; pU = []; qU = []
    for k, (A, p, q) in enumerate(cons):
        pc = jnp.clip(p, -CLIP, CLIP)
        qc = jnp.clip(q, -CLIP, CLIP)
        pc = jnp.where(jnp.isnan(pc), 0.0, pc)
        qc = jnp.where(jnp.isnan(qc), 0.0, qc)
        pos = A > 0.0
        neg = A < 0.0
        pL.append(jnp.where(pos, pc, z))
        qL.append(jnp.where(pos, qc, -BIGF))
        pU.append(jnp.where(neg, pc, z))
        qU.append(jnp.where(neg, qc, BIGF))
    yA = jnp.full_like(det, -BIGF)
    yB = jnp.full_like(det, BIGF)
    for i in range(5):
        for j in range(5):
            dp = pL[i] - pU[j]
            dq = qU[j] - qL[i]
            t = dq / dp
            yB = jnp.where(dp > 0.0, jnp.minimum(yB, t), yB)
            yA = jnp.where(dp < 0.0, jnp.maximum(yA, t), yA)
            yA = jnp.where((dp == 0.0) & (dq < 0.0), BIGF, yA)
    qlo = jnp.clip((S * yA + (S - 1.0)) * 0.5, -2.0, 300.0)
    qhi = jnp.clip((S * yB + (S - 1.0)) * 0.5, -2.0, 300.0)
    c_ref[15] = jnp.where(valid, qlo, 300.0)
    c_ref[26] = jnp.where(valid, qhi, -2.0)
    for i in (27, 28, 29, 30, 31, 42, 43, 44, 45, 46, 47):
        c_ref[i] = z


def _sc_raster_body(coef_hbm, out_hbm, cvm, den, idx):
    cid = lax.axis_index("c")
    sid = lax.axis_index("s")
    wid = sid * 2 + cid
    b = wid >> 3
    rbase = wid & 7

    pltpu.sync_copy(coef_hbm.at[b], cvm)

    def _init(r, _):
        for k in range(S2 // 16):
            col = k * 16
            den[r, pl.ds(col, 16)] = jnp.full((16,), 1.0 / FAR, jnp.float32)
            idx[r, pl.ds(col, 16)] = jnp.full((16,), -1, jnp.int32)
        return 0
    lax.fori_loop(0, BAND, _init, 0)

    lane = lax.iota(jnp.int32, 16)
    lane_f = lane.astype(jnp.float32)
    rbase_f = rbase.astype(jnp.float32)
    inv_s = jnp.float32(1.0 / S)

    def face_body(f, _):
        v1 = cvm[pl.ds(pl.multiple_of(f * NCOEF, 16), 16)]
        v2 = cvm[pl.ds(pl.multiple_of(f * NCOEF + 16, 16), 16)]
        v3 = cvm[pl.ds(pl.multiple_of(f * NCOEF + 32, 16), 16)]
        a0 = v1[0]; b0 = v1[1]; c0 = v1[2]
        a1 = v1[3]; b1 = v1[4]; c1 = v1[5]
        a2 = v1[6]; b2 = v1[7]; c2 = v1[8]
        ga = v1[9]; gb = v1[10]; gc = v1[11]
        radet = v1[12]; glo = v1[13]; ghi = v1[14]
        qlo = v1[15]; qhi = v2[10]

        rlo_g = jnp.maximum(qlo.astype(jnp.int32) - 1, 0)
        rhi_g = jnp.minimum(qhi.astype(jnp.int32) + 1, S - 1)
        tlo = jnp.maximum((rlo_g - rbase + 7) >> 3, 0)
        thi = jnp.minimum((rhi_g - rbase) >> 3, BAND - 1)

        @pl.when(tlo <= thi)
        def _do_face():
            @plsc.parallel_loop(tlo, thi + 1)
            def row_body(r):
                yr = (2.0 * (rbase_f + 8.0 * r.astype(jnp.float32))
                      + (1.0 - S)) * inv_s
                xlo = v2[5] + v2[0] * yr
                xhi = v3[5] + v3[0] * yr
                xlo = jnp.maximum(xlo, v2[6] + v2[1] * yr)
                xhi = jnp.minimum(xhi, v3[6] + v3[1] * yr)
                xlo = jnp.maximum(xlo, v2[7] + v2[2] * yr)
                xhi = jnp.minimum(xhi, v3[7] + v3[2] * yr)
                xlo = jnp.maximum(xlo, v2[8] + v2[3] * yr)
                xhi = jnp.minimum(xhi, v3[8] + v3[3] * yr)
                xlo = jnp.maximum(xlo, v2[9] + v2[4] * yr)
                xhi = jnp.minimum(xhi, v3[9] + v3[4] * yr)
                qjl = jnp.clip((S * xlo + (S - 1.0)) * 0.5, -2.0, 300.0)
                qjh = jnp.clip((S * xhi + (S - 1.0)) * 0.5, -2.0, 300.0)
                jl = jnp.maximum(qjl.astype(jnp.int32) - 1, 0)
                jh = jnp.minimum(qjh.astype(jnp.int32) + 1, S - 1)

                @pl.when(jl <= jh)
                def _do_row():
                    b0r = b0 * yr + c0
                    b1r = b1 * yr + c1
                    b2r = b2 * yr + c2
                    bgr = gb * yr + gc
                    base = jl & (-16)
                    nch = ((jh - base) >> 5) + 1

                    @plsc.parallel_loop(0, nch)
                    def ch_body(k):
                        c32 = base + k * 32
                        for h in range(2):
                            col = pl.multiple_of(c32 + h * 16, 16)
                            iv = lane + col
                            xv = (2.0 * iv.astype(jnp.float32)
                                  + (1.0 - S)) * inv_s
                            w0 = a0 * xv + b0r
                            w1 = a1 * xv + b1r
                            w2 = a2 * xv + b2r
                            g = ga * xv + bgr
                            dn = g * radet
                            dold = den[r, pl.ds(col, 16)]
                            m = ((w0 >= 0.0) & (w1 >= 0.0) & (w2 >= 0.0)
                                 & (g > glo) & (g < ghi) & (dn > dold))
                            if h == 1:
                                m = m & (iv < S)
                            den[r, pl.ds(col, 16)] = jnp.where(m, dn, dold)
                            iold = idx[r, pl.ds(col, 16)]
                            idx[r, pl.ds(col, 16)] = jnp.where(m, f, iold)

        return 0

    lax.fori_loop(0, F, face_body, 0)

    pltpu.sync_copy(idx.at[:, pl.ds(0, S)], out_hbm.at[b, rbase])


def kernel(inputs):
    B = inputs.shape[0]
    v = jnp.transpose(inputs, (3, 2, 0, 1)).reshape(9, B, F)
    coef = pl.pallas_call(
        _coef_body,
        out_shape=jax.ShapeDtypeStruct((NCOEF, B, F), jnp.float32),
    )(v)
    coef = jnp.transpose(coef, (1, 2, 0)).reshape(B, F * NCOEF)

    mesh = plsc.VectorSubcoreMesh(core_axis_name="c", subcore_axis_name="s")
    sc = functools.partial(
        pl.kernel,
        mesh=mesh,
        out_type=jax.ShapeDtypeStruct((B, 8, BAND, S), jnp.int32),
        scratch_types=[
            pltpu.VMEM((F * NCOEF,), jnp.float32),
            pltpu.VMEM((BAND, S2), jnp.float32),
            pltpu.VMEM((BAND, S2), jnp.int32),
        ],
    )(_sc_raster_body)
    out_perm = sc(coef)
    return jnp.transpose(out_perm, (0, 2, 1, 3)).reshape(B, S, S)
